# Initial kernel scaffold; baseline (speedup 1.0000x reference)
import jax
import jax.numpy as jnp
from jax.experimental import pallas as pl


def kernel(pos, centroids):
    raise NotImplementedError("write your pallas kernel here")


# speedup vs baseline: 4.7399x; 4.7399x over previous
import jax
import jax.numpy as jnp
from jax import lax
from jax.experimental import pallas as pl
from jax.experimental.pallas import tpu as pltpu
from jax.experimental.pallas import tpu_sc as plsc

_RADIUS2 = 0.15 ** 2
_K = 64
_B, _N = 4, 4096
_NW = 32
_ROWS_PER_W = (_B * _N) // _NW
_W_PER_BATCH = _N // _ROWS_PER_W
_NV = _N // 16
_BUF = 80


def _round_bf16(v):
    bits = lax.bitcast_convert_type(v, jnp.uint32)
    rb = bits + jnp.uint32(0x7FFF) + ((bits >> 16) & jnp.uint32(1))
    rb = rb & jnp.uint32(0xFFFF0000)
    return lax.bitcast_convert_type(rb, jnp.float32)


def _tec_body(pos_hbm, out_hbm, x, y, z, xb, yb, zb, n2, buf, stage):
    cid = lax.axis_index("c")
    sid = lax.axis_index("s")
    wid = sid * 2 + cid
    b = wid // _W_PER_BATCH
    rbase = (wid % _W_PER_BATCH) * _ROWS_PER_W

    pltpu.sync_copy(pos_hbm.at[pl.ds(b * 3 * _N, _N)], x)
    pltpu.sync_copy(pos_hbm.at[pl.ds(b * 3 * _N + _N, _N)], y)
    pltpu.sync_copy(pos_hbm.at[pl.ds(b * 3 * _N + 2 * _N, _N)], z)
    iota = lax.iota(jnp.int32, 16)
    ones = jnp.ones((16,), jnp.int32)

    def n2_body(v, carry):
        sl = pl.ds(v * 16, 16)
        xv = x[sl]
        yv = y[sl]
        zv = z[sl]
        n2[sl] = (xv * xv + yv * yv) + zv * zv
        xb[sl] = _round_bf16(xv)
        yb[sl] = _round_bf16(yv)
        zb[sl] = _round_bf16(zv)
        return carry

    lax.fori_loop(0, _NV, n2_body, 0)

    def row_body(r, carry):
        row = jnp.full((16,), rbase + r, jnp.int32)
        m2x = -2.0 * plsc.load_gather(xb, [row])
        m2y = -2.0 * plsc.load_gather(yb, [row])
        m2z = -2.0 * plsc.load_gather(zb, [row])
        qn = plsc.load_gather(n2, [row])

        def j_body(v, base):
            sl = pl.ds(v * 16, 16)
            px = xb[sl]
            py = yb[sl]
            pz = zb[sl]
            t = (((px * m2x + py * m2y) + pz * m2z) + qn) + n2[sl]
            mask = t <= _RADIUS2
            incl = plsc.cumsum(ones, mask=mask)
            pos = jnp.minimum(base + incl - 1, _BUF - 1)
            plsc.store_scatter(buf, [pos], iota + v * 16, mask=mask)
            return base + plsc.all_reduce_population_count(mask)

        base = lax.fori_loop(0, _NV, j_body, jnp.zeros((16,), jnp.int32))
        cnt = jnp.minimum(base, _K)
        cur0 = buf[pl.ds(0, 16)]
        first = jnp.broadcast_to(cur0[0], (16,))
        for v in range(_K // 16):
            cur = cur0 if v == 0 else buf[pl.ds(v * 16, 16)]
            keep = (iota + v * 16) < cnt
            stage[pl.ds(r * _K + v * 16, 16)] = jnp.where(keep, cur, first)
        return carry

    lax.fori_loop(0, _ROWS_PER_W, row_body, 0)
    pltpu.sync_copy(
        stage, out_hbm.at[pl.ds(wid * _ROWS_PER_W * _K, _ROWS_PER_W * _K)]
    )


def _build_kernel():
    mesh = plsc.VectorSubcoreMesh(core_axis_name="c", subcore_axis_name="s")
    return pl.kernel(
        _tec_body,
        out_type=jax.ShapeDtypeStruct((_B * _N * _K,), jnp.int32),
        mesh=mesh,
        scratch_types=[
            pltpu.VMEM((_N,), jnp.float32),
            pltpu.VMEM((_N,), jnp.float32),
            pltpu.VMEM((_N,), jnp.float32),
            pltpu.VMEM((_N,), jnp.float32),
            pltpu.VMEM((_N,), jnp.float32),
            pltpu.VMEM((_N,), jnp.float32),
            pltpu.VMEM((_N,), jnp.float32),
            pltpu.VMEM((_BUF,), jnp.int32),
            pltpu.VMEM((_ROWS_PER_W * _K,), jnp.int32),
        ],
        compiler_params=pltpu.CompilerParams(needs_layout_passes=False),
    )


@jax.jit
def kernel(pos, centroids):
    del centroids
    posT = jnp.transpose(pos, (0, 2, 1))
    out = _build_kernel()(posT.reshape(-1))
    return out.reshape(_B, _N, _K)


# speedup vs baseline: 16.6310x; 3.5087x over previous
import jax
import jax.numpy as jnp
from jax import lax
from jax.experimental import pallas as pl
from jax.experimental.pallas import tpu as pltpu
from jax.experimental.pallas import tpu_sc as plsc

_RADIUS2 = 0.15 ** 2
_K = 64
_B, _N = 4, 4096
_NW = 32
_ROWS_PER_W = (_B * _N) // _NW
_W_PER_BATCH = _N // _ROWS_PER_W
_NV = _N // 16
_R = 4
_BUF = 80


def _round_bf16(v):
    bits = lax.bitcast_convert_type(v, jnp.uint32)
    rb = bits + jnp.uint32(0x7FFF) + ((bits >> 16) & jnp.uint32(1))
    rb = rb & jnp.uint32(0xFFFF0000)
    return lax.bitcast_convert_type(rb, jnp.float32)


def _tec_body(pos_hbm, out_hbm, x, y, z, xb, yb, zb, n2, buf, stage):
    cid = lax.axis_index("c")
    sid = lax.axis_index("s")
    wid = sid * 2 + cid
    b = wid // _W_PER_BATCH
    rbase = (wid % _W_PER_BATCH) * _ROWS_PER_W

    pltpu.sync_copy(pos_hbm.at[pl.ds(b * 3 * _N, _N)], x)
    pltpu.sync_copy(pos_hbm.at[pl.ds(b * 3 * _N + _N, _N)], y)
    pltpu.sync_copy(pos_hbm.at[pl.ds(b * 3 * _N + 2 * _N, _N)], z)
    iota = lax.iota(jnp.int32, 16)
    ones = jnp.ones((16,), jnp.int32)

    def n2_body(v, carry):
        sl = pl.ds(v * 16, 16)
        xv = x[sl]
        yv = y[sl]
        zv = z[sl]
        n2[sl] = (xv * xv + yv * yv) + zv * zv
        xb[sl] = _round_bf16(xv)
        yb[sl] = _round_bf16(yv)
        zb[sl] = _round_bf16(zv)
        return carry

    lax.fori_loop(0, _NV, n2_body, 0)

    def group_body(g, carry):
        r0 = rbase + g * _R
        m2xs, m2ys, m2zs, qns = [], [], [], []
        for i in range(_R):
            ri = jnp.full((16,), r0 + i, jnp.int32)
            m2xs.append(-2.0 * plsc.load_gather(xb, [ri]))
            m2ys.append(-2.0 * plsc.load_gather(yb, [ri]))
            m2zs.append(-2.0 * plsc.load_gather(zb, [ri]))
            qns.append(plsc.load_gather(n2, [ri]))

        def j_body(v, bases):
            sl = pl.ds(v * 16, 16)
            px = xb[sl]
            py = yb[sl]
            pz = zb[sl]
            nn = n2[sl]
            jv = iota + v * 16
            new_bases = []
            for i in range(_R):
                t = (((px * m2xs[i] + py * m2ys[i]) + pz * m2zs[i])
                     + qns[i]) + nn
                mask = t <= _RADIUS2
                incl = plsc.cumsum(ones, mask=mask)
                pos = jnp.minimum(bases[i] + incl - 1, _BUF - 1)
                plsc.store_scatter(buf, [pos + i * _BUF], jv, mask=mask)
                new_bases.append(
                    bases[i] + plsc.all_reduce_population_count(mask))
            return tuple(new_bases)

        bases = lax.fori_loop(
            0, _NV, j_body,
            tuple(jnp.zeros((16,), jnp.int32) for _ in range(_R)))
        for i in range(_R):
            cnt = bases[i]
            cur0 = buf[pl.ds(i * _BUF, 16)]
            first = jnp.broadcast_to(cur0[0], (16,))
            for v in range(_K // 16):
                cur = cur0 if v == 0 else buf[pl.ds(i * _BUF + v * 16, 16)]
                keep = (iota + v * 16) < cnt
                stage[pl.ds((g * _R + i) * _K + v * 16, 16)] = jnp.where(
                    keep, cur, first)
        return carry

    lax.fori_loop(0, _ROWS_PER_W // _R, group_body, 0)
    pltpu.sync_copy(
        stage, out_hbm.at[pl.ds(wid * _ROWS_PER_W * _K, _ROWS_PER_W * _K)]
    )


def _build_kernel():
    mesh = plsc.VectorSubcoreMesh(core_axis_name="c", subcore_axis_name="s")
    return pl.kernel(
        _tec_body,
        out_type=jax.ShapeDtypeStruct((_B * _N * _K,), jnp.int32),
        mesh=mesh,
        scratch_types=[
            pltpu.VMEM((_N,), jnp.float32),
            pltpu.VMEM((_N,), jnp.float32),
            pltpu.VMEM((_N,), jnp.float32),
            pltpu.VMEM((_N,), jnp.float32),
            pltpu.VMEM((_N,), jnp.float32),
            pltpu.VMEM((_N,), jnp.float32),
            pltpu.VMEM((_N,), jnp.float32),
            pltpu.VMEM((_R * _BUF,), jnp.int32),
            pltpu.VMEM((_ROWS_PER_W * _K,), jnp.int32),
        ],
        compiler_params=pltpu.CompilerParams(needs_layout_passes=False),
    )


@jax.jit
def kernel(pos, centroids):
    del centroids
    posT = jnp.transpose(pos, (0, 2, 1))
    out = _build_kernel()(posT.reshape(-1))
    return out.reshape(_B, _N, _K)


# speedup vs baseline: 26.8879x; 1.6167x over previous
import jax
import jax.numpy as jnp
from jax import lax
from jax.experimental import pallas as pl
from jax.experimental.pallas import tpu as pltpu
from jax.experimental.pallas import tpu_sc as plsc

_RADIUS2 = 0.15 ** 2
_K = 64
_B, _N = 4, 4096
_NW = 32
_ROWS_PER_W = (_B * _N) // _NW
_W_PER_BATCH = _N // _ROWS_PER_W
_NV = _N // 16
_R = 8
_BUF = _N


def _round_bf16(v):
    bits = lax.bitcast_convert_type(v, jnp.uint32)
    rb = bits + jnp.uint32(0x7FFF) + ((bits >> 16) & jnp.uint32(1))
    rb = rb & jnp.uint32(0xFFFF0000)
    return lax.bitcast_convert_type(rb, jnp.float32)


def _tec_body(pos_hbm, out_hbm, x, y, z, xb, yb, zb, n2, buf, stage):
    cid = lax.axis_index("c")
    sid = lax.axis_index("s")
    wid = sid * 2 + cid
    b = wid // _W_PER_BATCH
    rbase = (wid % _W_PER_BATCH) * _ROWS_PER_W

    pltpu.sync_copy(pos_hbm.at[pl.ds(b * 3 * _N, _N)], x)
    pltpu.sync_copy(pos_hbm.at[pl.ds(b * 3 * _N + _N, _N)], y)
    pltpu.sync_copy(pos_hbm.at[pl.ds(b * 3 * _N + 2 * _N, _N)], z)
    iota = lax.iota(jnp.int32, 16)
    ones = jnp.ones((16,), jnp.int32)

    def n2_body(v, carry):
        sl = pl.ds(v * 16, 16)
        xv = x[sl]
        yv = y[sl]
        zv = z[sl]
        n2[sl] = (xv * xv + yv * yv) + zv * zv
        xb[sl] = _round_bf16(xv)
        yb[sl] = _round_bf16(yv)
        zb[sl] = _round_bf16(zv)
        return carry

    lax.fori_loop(0, _NV, n2_body, 0)

    def group_body(g, carry):
        r0 = rbase + g * _R
        m2xs, m2ys, m2zs, qns = [], [], [], []
        for i in range(_R):
            ri = jnp.full((16,), r0 + i, jnp.int32)
            m2xs.append(-2.0 * plsc.load_gather(xb, [ri]))
            m2ys.append(-2.0 * plsc.load_gather(yb, [ri]))
            m2zs.append(-2.0 * plsc.load_gather(zb, [ri]))
            qns.append(plsc.load_gather(n2, [ri]))

        def j_body(v, bases):
            sl = pl.ds(v * 16, 16)
            px = xb[sl]
            py = yb[sl]
            pz = zb[sl]
            nn = n2[sl]
            jv = iota + v * 16
            new_bases = []
            for i in range(_R):
                t = (((px * m2xs[i] + py * m2ys[i]) + pz * m2zs[i])
                     + qns[i]) + nn
                mask = t <= _RADIUS2
                incl = plsc.cumsum(ones, mask=mask)
                plsc.store_scatter(buf, [bases[i] + incl], jv, mask=mask)
                new_bases.append(
                    bases[i] + plsc.all_reduce_population_count(mask))
            return tuple(new_bases)

        bases = lax.fori_loop(
            0, _NV, j_body,
            tuple(jnp.full((16,), i * _BUF - 1, jnp.int32)
                  for i in range(_R)))
        for i in range(_R):
            cnt = bases[i] - (i * _BUF - 1)
            cur0 = buf[pl.ds(i * _BUF, 16)]
            first = jnp.broadcast_to(cur0[0], (16,))
            for v in range(_K // 16):
                cur = cur0 if v == 0 else buf[pl.ds(i * _BUF + v * 16, 16)]
                keep = (iota + v * 16) < cnt
                stage[pl.ds((g * _R + i) * _K + v * 16, 16)] = jnp.where(
                    keep, cur, first)
        return carry

    lax.fori_loop(0, _ROWS_PER_W // _R, group_body, 0)
    pltpu.sync_copy(
        stage, out_hbm.at[pl.ds(wid * _ROWS_PER_W * _K, _ROWS_PER_W * _K)]
    )


def _build_kernel():
    mesh = plsc.VectorSubcoreMesh(core_axis_name="c", subcore_axis_name="s")
    return pl.kernel(
        _tec_body,
        out_type=jax.ShapeDtypeStruct((_B * _N * _K,), jnp.int32),
        mesh=mesh,
        scratch_types=[
            pltpu.VMEM((_N,), jnp.float32),
            pltpu.VMEM((_N,), jnp.float32),
            pltpu.VMEM((_N,), jnp.float32),
            pltpu.VMEM((_N,), jnp.float32),
            pltpu.VMEM((_N,), jnp.float32),
            pltpu.VMEM((_N,), jnp.float32),
            pltpu.VMEM((_N,), jnp.float32),
            pltpu.VMEM((_R * _BUF,), jnp.int32),
            pltpu.VMEM((_ROWS_PER_W * _K,), jnp.int32),
        ],
        compiler_params=pltpu.CompilerParams(needs_layout_passes=False),
    )


@jax.jit
def kernel(pos, centroids):
    del centroids
    posT = jnp.transpose(pos, (0, 2, 1))
    out = _build_kernel()(posT.reshape(-1))
    return out.reshape(_B, _N, _K)
